# baseline (speedup 1.0000x reference)
import functools

import jax
import jax.numpy as jnp
from jax import lax
from jax.experimental import pallas as pl
from jax.experimental.pallas import tpu as pltpu
from jax.experimental.pallas import tpu_sc as plsc

_G = 128
_S = 32
_NP = 128
_CH = 4096


def _tc_kernel(xyz8_ref, xyzT_ref, xP_ref, yP_ref, zP_ref,
               fidx_ref, idx_ref, cattr_ref, d2_ref, cattr_s_ref):
    f32 = jnp.float32
    n = xyzT_ref.shape[2]
    nc = n // _CH
    xP = xP_ref[0]
    yP = yP_ref[0]
    zP = zP_ref[0]

    r_iota = jax.lax.broadcasted_iota(jnp.int32, (_NP, _NP), 0)
    c_iota = jax.lax.broadcasted_iota(jnp.int32, (_NP, _NP), 1)
    lin = r_iota * _NP + c_iota
    g_iota = jax.lax.broadcasted_iota(jnp.int32, (1, _G), 1)

    def fps_body(i, carry):
        dist, far, idx_row = carry
        row = xyz8_ref[0, pl.ds(far, 1), :]
        cattr_s_ref[pl.ds(i, 1), :] = row
        idx_row = jnp.where(g_iota == i, far, idx_row)
        cx = row[0:1, 0:1]
        cy = row[0:1, 1:2]
        cz = row[0:1, 2:3]
        dx = xP - cx
        dy = yP - cy
        dz = zP - cz
        d = dx * dx + dy * dy
        d = d + dz * dz
        dist = jnp.minimum(dist, d)
        m = jnp.max(dist)
        far2 = jnp.min(jnp.where(dist == m, lin, jnp.int32(n)))
        return dist, far2, idx_row

    dist0 = jnp.full((_NP, _NP), 1e10, dtype=f32)
    idx0 = jnp.zeros((1, _G), dtype=jnp.int32)
    _, _, idx_row = jax.lax.fori_loop(
        0, _G, fps_body, (dist0, jnp.int32(0), idx0))
    idx_ref[0] = idx_row

    cattr = cattr_s_ref[...]
    cattr_ref[0] = cattr
    c2 = jnp.sum(cattr * cattr, axis=1, keepdims=True)

    def d2_chunk(c, _):
        xTc = xyzT_ref[0, :, pl.ds(c * _CH, _CH)]
        n2c = jnp.sum(xTc * xTc, axis=0, keepdims=True)
        xyc = jax.lax.dot_general(cattr, xTc, (((1,), (0,)), ((), ())),
                                  preferred_element_type=f32)
        d2_ref[:, pl.ds(c * _CH, _CH)] = (c2 + n2c) - 2.0 * xyc
        return 0

    jax.lax.fori_loop(0, nc, d2_chunk, 0)

    iota_ch = jax.lax.broadcasted_iota(jnp.int32, (1, _CH), 1)
    nc_iota = jax.lax.broadcasted_iota(jnp.int32, (1, nc), 1)
    s_iota = jax.lax.broadcasted_iota(jnp.int32, (1, _S), 1)

    def topk_body(s, carry):
        idx32, vprev, iprev = carry

        def chunk_pass(c, cc):
            mins, cands = cc
            off = c * _CH
            gidx = iota_ch + off
            d2c = d2_ref[:, pl.ds(off, _CH)]
            live = (d2c > vprev) | ((d2c == vprev) & (gidx > iprev))
            d2m = jnp.where(live, d2c, jnp.inf)
            mc = jnp.min(d2m, axis=1, keepdims=True)
            candc = jnp.min(
                jnp.where(d2m == mc, gidx, jnp.int32(n)),
                axis=1, keepdims=True)
            mins = jnp.where(nc_iota == c, mc, mins)
            cands = jnp.where(nc_iota == c, candc, cands)
            return mins, cands

        mins0 = jnp.full((_G, nc), jnp.inf, dtype=f32)
        cands0 = jnp.full((_G, nc), n, dtype=jnp.int32)
        mins, cands = jax.lax.fori_loop(0, nc, chunk_pass, (mins0, cands0))
        mval = jnp.min(mins, axis=1, keepdims=True)
        midx = jnp.min(jnp.where(mins == mval, cands, jnp.int32(n)),
                       axis=1, keepdims=True)
        idx32 = jnp.where(s_iota == s, midx, idx32)
        return idx32, mval, midx

    idx32_0 = jnp.zeros((_G, _S), dtype=jnp.int32)
    idx32, _, _ = jax.lax.fori_loop(
        0, _S, topk_body,
        (idx32_0, jnp.full((_G, 1), -jnp.inf, f32),
         jnp.full((_G, 1), -1, jnp.int32)))
    fidx_ref[0] = idx32 + pl.program_id(0) * n


def _sc_gather_make(rows_total, nw, gpw, rpw):
    mesh = plsc.VectorSubcoreMesh(core_axis_name="c", subcore_axis_name="s")
    nchunk = rpw // 128

    @functools.partial(
        pl.kernel, mesh=mesh,
        compiler_params=pltpu.CompilerParams(use_tc_tiling_on_sc=False),
        out_type=jax.ShapeDtypeStruct((rows_total, 16), jnp.float32),
        scratch_types=[
            pltpu.VMEM((nchunk, 128), jnp.int32),
            pltpu.VMEM((rpw, 16), jnp.float32),
            pltpu.VMEM((gpw, 16), jnp.float32),
            pltpu.SemaphoreType.DMA,
        ],
    )
    def sc_gather(xyz16_hbm, fidx_hbm, cattr16_hbm, out_hbm,
                  idx_v, rows_v, cent_v, sem):
        nc_ = plsc.get_sparse_core_info().num_cores
        wid = lax.axis_index("s") * nc_ + lax.axis_index("c")
        rbase = wid * rpw
        gbase = wid * gpw
        pltpu.sync_copy(fidx_hbm.at[wid], idx_v)
        pltpu.sync_copy(cattr16_hbm.at[pl.ds(gbase, gpw)], cent_v)
        copies = []
        for c in range(nchunk):
            copies.append(pltpu.async_copy(
                xyz16_hbm.at[idx_v.at[c]],
                rows_v.at[pl.ds(c * 128, 128)], sem))
        for cp in copies:
            cp.wait()

        lane = jnp.arange(16, dtype=jnp.int32)
        zero = jnp.zeros((16,), dtype=jnp.float32)

        def group_body(g, _):
            cmask = jnp.where(lane < 3, cent_v[g], zero)
            base = g * _S
            for s in range(_S):
                rows_v[base + s] = rows_v[base + s] - cmask
            return 0

        lax.fori_loop(0, gpw, group_body, 0)
        pltpu.sync_copy(rows_v, out_hbm.at[pl.ds(rbase, rpw)])

    return sc_gather


@jax.jit
def kernel(xyz):
    b, n, a = xyz.shape
    f32 = jnp.float32
    xyz8 = jnp.concatenate(
        [xyz, jnp.zeros((b, n, 8 - a), dtype=xyz.dtype)], axis=-1)
    xyzT = xyz8.transpose(0, 2, 1)
    planes = xyz[:, :, :3].transpose(0, 2, 1).reshape(b, 3, _NP, n // _NP)

    fidx, idx, cattr = pl.pallas_call(
        _tc_kernel,
        grid=(b,),
        in_specs=[
            pl.BlockSpec((1, n, 8), lambda i: (i, 0, 0)),
            pl.BlockSpec((1, 8, n), lambda i: (i, 0, 0)),
            pl.BlockSpec((1, _NP, n // _NP), lambda i: (i, 0, 0)),
            pl.BlockSpec((1, _NP, n // _NP), lambda i: (i, 0, 0)),
            pl.BlockSpec((1, _NP, n // _NP), lambda i: (i, 0, 0)),
        ],
        out_specs=[
            pl.BlockSpec((1, _G, _S), lambda i: (i, 0, 0)),
            pl.BlockSpec((1, 1, _G), lambda i: (i, 0, 0)),
            pl.BlockSpec((1, _G, 8), lambda i: (i, 0, 0)),
        ],
        out_shape=[
            jax.ShapeDtypeStruct((b, _G, _S), jnp.int32),
            jax.ShapeDtypeStruct((b, 1, _G), jnp.int32),
            jax.ShapeDtypeStruct((b, _G, 8), f32),
        ],
        scratch_shapes=[
            pltpu.VMEM((_G, n), f32),
            pltpu.VMEM((_G, 8), f32),
        ],
        compiler_params=pltpu.CompilerParams(
            dimension_semantics=("arbitrary",),
        ),
    )(xyz8, xyzT, planes[:, 0], planes[:, 1], planes[:, 2])

    rows_total = b * _G * _S
    nw = 32
    rpw = rows_total // nw
    gpw = (b * _G) // nw
    xyz16 = jnp.concatenate(
        [xyz.reshape(b * n, a),
         jnp.zeros((b * n, 16 - a), dtype=f32)], axis=-1)
    cattr16 = jnp.concatenate(
        [cattr.reshape(b * _G, 8),
         jnp.zeros((b * _G, 8), dtype=f32)], axis=-1)
    fidx3 = fidx.reshape(nw, rpw // 128, 128)

    nb16 = _sc_gather_make(rows_total, nw, gpw, rpw)(xyz16, fidx3, cattr16)

    neighborhood = nb16.reshape(b, _G, _S, 16)[..., :a]
    center_idx = idx.reshape(b, _G)
    centroids_attrs = cattr[..., :a]
    return (neighborhood, center_idx, centroids_attrs,
            centroids_attrs[..., :3])


# speedup vs baseline: 1.1653x; 1.1653x over previous
import functools

import jax
import jax.numpy as jnp
from jax import lax
from jax.experimental import pallas as pl
from jax.experimental.pallas import tpu as pltpu
from jax.experimental.pallas import tpu_sc as plsc

_G = 128
_S = 32
_NP = 128
_CH = 4096


def _tc_kernel(xyz8_ref, xyzT_ref, xP_ref, yP_ref, zP_ref,
               fidx_ref, idx_ref, cattr_ref, d2_ref, cattr_s_ref):
    f32 = jnp.float32
    n = xyzT_ref.shape[2]
    nc = n // _CH
    xP = xP_ref[0]
    yP = yP_ref[0]
    zP = zP_ref[0]

    r_iota = jax.lax.broadcasted_iota(jnp.int32, (_NP, _NP), 0)
    c_iota = jax.lax.broadcasted_iota(jnp.int32, (_NP, _NP), 1)
    lin = r_iota * _NP + c_iota
    g_iota = jax.lax.broadcasted_iota(jnp.int32, (1, _G), 1)

    def fps_body(i, carry):
        dist, far, idx_row = carry
        row = xyz8_ref[0, pl.ds(far, 1), :]
        cattr_s_ref[pl.ds(i, 1), :] = row
        idx_row = jnp.where(g_iota == i, far, idx_row)
        cx = row[0:1, 0:1]
        cy = row[0:1, 1:2]
        cz = row[0:1, 2:3]
        dx = xP - cx
        dy = yP - cy
        dz = zP - cz
        d = dx * dx + dy * dy
        d = d + dz * dz
        dist = jnp.minimum(dist, d)
        m = jnp.max(dist)
        far2 = jnp.min(jnp.where(dist == m, lin, jnp.int32(n)))
        return dist, far2, idx_row

    dist0 = jnp.full((_NP, _NP), 1e10, dtype=f32)
    idx0 = jnp.zeros((1, _G), dtype=jnp.int32)
    _, _, idx_row = jax.lax.fori_loop(
        0, _G, fps_body, (dist0, jnp.int32(0), idx0))
    idx_ref[0] = idx_row

    cattr = cattr_s_ref[...]
    cattr_ref[0] = cattr
    c2 = jnp.sum(cattr * cattr, axis=1, keepdims=True)

    def d2_chunk(c, _):
        xTc = xyzT_ref[0, :, pl.ds(c * _CH, _CH)]
        n2c = jnp.sum(xTc * xTc, axis=0, keepdims=True)
        xyc = jax.lax.dot_general(cattr, xTc, (((1,), (0,)), ((), ())),
                                  preferred_element_type=f32)
        d2_ref[:, pl.ds(c * _CH, _CH)] = (c2 + n2c) - 2.0 * xyc
        return 0

    jax.lax.fori_loop(0, nc, d2_chunk, 0)

    iota_ch = jax.lax.broadcasted_iota(jnp.int32, (1, _CH), 1)
    nc_iota = jax.lax.broadcasted_iota(jnp.int32, (1, nc), 1)
    s_iota = jax.lax.broadcasted_iota(jnp.int32, (1, _S), 1)

    def topk_body(s, carry):
        idx32, midx_prev = carry

        def chunk_pass(c, cc):
            mins, cands = cc
            off = c * _CH
            d2c = d2_ref[:, pl.ds(off, _CH)]
            d2c = jnp.where(iota_ch + off == midx_prev,
                            jnp.float32(1e30), d2c)
            d2_ref[:, pl.ds(off, _CH)] = d2c
            mc = jnp.min(d2c, axis=1, keepdims=True)
            candc = jnp.min(
                jnp.where(d2c == mc, iota_ch + off, jnp.int32(n)),
                axis=1, keepdims=True)
            mins = jnp.where(nc_iota == c, mc, mins)
            cands = jnp.where(nc_iota == c, candc, cands)
            return mins, cands

        mins0 = jnp.full((_G, nc), jnp.inf, dtype=f32)
        cands0 = jnp.full((_G, nc), n, dtype=jnp.int32)
        mins, cands = jax.lax.fori_loop(0, nc, chunk_pass, (mins0, cands0))
        mval = jnp.min(mins, axis=1, keepdims=True)
        midx = jnp.min(jnp.where(mins == mval, cands, jnp.int32(n)),
                       axis=1, keepdims=True)
        idx32 = jnp.where(s_iota == s, midx, idx32)
        return idx32, midx

    idx32_0 = jnp.zeros((_G, _S), dtype=jnp.int32)
    idx32, _ = jax.lax.fori_loop(
        0, _S, topk_body, (idx32_0, jnp.full((_G, 1), -1, jnp.int32)))
    fidx_ref[0] = idx32 + pl.program_id(0) * n


def _sc_gather_make(rows_total, nw, gpw, rpw):
    mesh = plsc.VectorSubcoreMesh(core_axis_name="c", subcore_axis_name="s")
    nchunk = rpw // 128

    @functools.partial(
        pl.kernel, mesh=mesh,
        compiler_params=pltpu.CompilerParams(use_tc_tiling_on_sc=False),
        out_type=jax.ShapeDtypeStruct((rows_total, 16), jnp.float32),
        scratch_types=[
            pltpu.VMEM((nchunk, 128), jnp.int32),
            pltpu.VMEM((rpw, 16), jnp.float32),
            pltpu.VMEM((gpw, 16), jnp.float32),
            pltpu.SemaphoreType.DMA,
        ],
    )
    def sc_gather(xyz16_hbm, fidx_hbm, cattr16_hbm, out_hbm,
                  idx_v, rows_v, cent_v, sem):
        nc_ = plsc.get_sparse_core_info().num_cores
        wid = lax.axis_index("s") * nc_ + lax.axis_index("c")
        rbase = wid * rpw
        gbase = wid * gpw
        pltpu.sync_copy(fidx_hbm.at[wid], idx_v)
        pltpu.sync_copy(cattr16_hbm.at[pl.ds(gbase, gpw)], cent_v)
        copies = []
        for c in range(nchunk):
            copies.append(pltpu.async_copy(
                xyz16_hbm.at[idx_v.at[c]],
                rows_v.at[pl.ds(c * 128, 128)], sem))
        for cp in copies:
            cp.wait()

        lane = jnp.arange(16, dtype=jnp.int32)
        zero = jnp.zeros((16,), dtype=jnp.float32)

        def group_body(g, _):
            cmask = jnp.where(lane < 3, cent_v[g], zero)
            base = g * _S
            for s in range(_S):
                rows_v[base + s] = rows_v[base + s] - cmask
            return 0

        lax.fori_loop(0, gpw, group_body, 0)
        pltpu.sync_copy(rows_v, out_hbm.at[pl.ds(rbase, rpw)])

    return sc_gather


@jax.jit
def kernel(xyz):
    b, n, a = xyz.shape
    f32 = jnp.float32
    xyz8 = jnp.concatenate(
        [xyz, jnp.zeros((b, n, 8 - a), dtype=xyz.dtype)], axis=-1)
    xyzT = xyz8.transpose(0, 2, 1)
    planes = xyz[:, :, :3].transpose(0, 2, 1).reshape(b, 3, _NP, n // _NP)

    fidx, idx, cattr = pl.pallas_call(
        _tc_kernel,
        grid=(b,),
        in_specs=[
            pl.BlockSpec((1, n, 8), lambda i: (i, 0, 0)),
            pl.BlockSpec((1, 8, n), lambda i: (i, 0, 0)),
            pl.BlockSpec((1, _NP, n // _NP), lambda i: (i, 0, 0)),
            pl.BlockSpec((1, _NP, n // _NP), lambda i: (i, 0, 0)),
            pl.BlockSpec((1, _NP, n // _NP), lambda i: (i, 0, 0)),
        ],
        out_specs=[
            pl.BlockSpec((1, _G, _S), lambda i: (i, 0, 0)),
            pl.BlockSpec((1, 1, _G), lambda i: (i, 0, 0)),
            pl.BlockSpec((1, _G, 8), lambda i: (i, 0, 0)),
        ],
        out_shape=[
            jax.ShapeDtypeStruct((b, _G, _S), jnp.int32),
            jax.ShapeDtypeStruct((b, 1, _G), jnp.int32),
            jax.ShapeDtypeStruct((b, _G, 8), f32),
        ],
        scratch_shapes=[
            pltpu.VMEM((_G, n), f32),
            pltpu.VMEM((_G, 8), f32),
        ],
        compiler_params=pltpu.CompilerParams(
            dimension_semantics=("arbitrary",),
        ),
    )(xyz8, xyzT, planes[:, 0], planes[:, 1], planes[:, 2])

    rows_total = b * _G * _S
    nw = 32
    rpw = rows_total // nw
    gpw = (b * _G) // nw
    xyz16 = jnp.concatenate(
        [xyz.reshape(b * n, a),
         jnp.zeros((b * n, 16 - a), dtype=f32)], axis=-1)
    cattr16 = jnp.concatenate(
        [cattr.reshape(b * _G, 8),
         jnp.zeros((b * _G, 8), dtype=f32)], axis=-1)
    fidx3 = fidx.reshape(nw, rpw // 128, 128)

    nb16 = _sc_gather_make(rows_total, nw, gpw, rpw)(xyz16, fidx3, cattr16)

    neighborhood = nb16.reshape(b, _G, _S, 16)[..., :a]
    center_idx = idx.reshape(b, _G)
    centroids_attrs = cattr[..., :a]
    return (neighborhood, center_idx, centroids_attrs,
            centroids_attrs[..., :3])


# speedup vs baseline: 1.1748x; 1.0081x over previous
import functools

import jax
import jax.numpy as jnp
from jax import lax
from jax.experimental import pallas as pl
from jax.experimental.pallas import tpu as pltpu
from jax.experimental.pallas import tpu_sc as plsc

_G = 128
_S = 32
_NP = 128
_CH = 8192


def _tc_kernel(xyz8_ref, xyzT_ref, xP_ref, yP_ref, zP_ref,
               fidx_ref, idx_ref, cattr_ref, d2_ref, cattr_s_ref):
    f32 = jnp.float32
    n = xyzT_ref.shape[2]
    nc = n // _CH
    xP = xP_ref[0]
    yP = yP_ref[0]
    zP = zP_ref[0]

    r_iota = jax.lax.broadcasted_iota(jnp.int32, (_NP, _NP), 0)
    c_iota = jax.lax.broadcasted_iota(jnp.int32, (_NP, _NP), 1)
    lin = r_iota * _NP + c_iota
    g_iota = jax.lax.broadcasted_iota(jnp.int32, (1, _G), 1)

    def fps_body(i, carry):
        dist, far, idx_row = carry
        row = xyz8_ref[0, pl.ds(far, 1), :]
        cattr_s_ref[pl.ds(i, 1), :] = row
        idx_row = jnp.where(g_iota == i, far, idx_row)
        cx = row[0:1, 0:1]
        cy = row[0:1, 1:2]
        cz = row[0:1, 2:3]
        dx = xP - cx
        dy = yP - cy
        dz = zP - cz
        d = dx * dx + dy * dy
        d = d + dz * dz
        dist = jnp.minimum(dist, d)
        m = jnp.max(dist)
        far2 = jnp.min(jnp.where(dist == m, lin, jnp.int32(n)))
        return dist, far2, idx_row

    dist0 = jnp.full((_NP, _NP), 1e10, dtype=f32)
    idx0 = jnp.zeros((1, _G), dtype=jnp.int32)
    _, _, idx_row = jax.lax.fori_loop(
        0, _G, fps_body, (dist0, jnp.int32(0), idx0))
    idx_ref[0] = idx_row

    cattr = cattr_s_ref[...]
    cattr_ref[0] = cattr
    c2 = jnp.sum(cattr * cattr, axis=1, keepdims=True)

    def d2_chunk(c, _):
        xTc = xyzT_ref[0, :, pl.ds(c * _CH, _CH)]
        n2c = jnp.sum(xTc * xTc, axis=0, keepdims=True)
        xyc = jax.lax.dot_general(cattr, xTc, (((1,), (0,)), ((), ())),
                                  preferred_element_type=f32)
        d2_ref[:, pl.ds(c * _CH, _CH)] = (c2 + n2c) - 2.0 * xyc
        return 0

    jax.lax.fori_loop(0, nc, d2_chunk, 0)

    iota_ch = jax.lax.broadcasted_iota(jnp.int32, (1, _CH), 1)
    nc_iota = jax.lax.broadcasted_iota(jnp.int32, (1, nc), 1)
    s_iota = jax.lax.broadcasted_iota(jnp.int32, (1, _S), 1)

    def topk_body(s, carry):
        idx32, midx_prev = carry

        def chunk_pass(c, cc):
            mins, cands = cc
            off = c * _CH
            d2c = d2_ref[:, pl.ds(off, _CH)]
            d2c = jnp.where(iota_ch + off == midx_prev,
                            jnp.float32(1e30), d2c)
            d2_ref[:, pl.ds(off, _CH)] = d2c
            mc = jnp.min(d2c, axis=1, keepdims=True)
            candc = jnp.min(
                jnp.where(d2c == mc, iota_ch + off, jnp.int32(n)),
                axis=1, keepdims=True)
            mins = jnp.where(nc_iota == c, mc, mins)
            cands = jnp.where(nc_iota == c, candc, cands)
            return mins, cands

        mins0 = jnp.full((_G, nc), jnp.inf, dtype=f32)
        cands0 = jnp.full((_G, nc), n, dtype=jnp.int32)
        mins, cands = jax.lax.fori_loop(0, nc, chunk_pass, (mins0, cands0))
        mval = jnp.min(mins, axis=1, keepdims=True)
        midx = jnp.min(jnp.where(mins == mval, cands, jnp.int32(n)),
                       axis=1, keepdims=True)
        idx32 = jnp.where(s_iota == s, midx, idx32)
        return idx32, midx

    idx32_0 = jnp.zeros((_G, _S), dtype=jnp.int32)
    idx32, _ = jax.lax.fori_loop(
        0, _S, topk_body, (idx32_0, jnp.full((_G, 1), -1, jnp.int32)))
    fidx_ref[0] = idx32 + pl.program_id(0) * n


def _sc_gather_make(rows_total, nw, gpw, rpw):
    mesh = plsc.VectorSubcoreMesh(core_axis_name="c", subcore_axis_name="s")
    nchunk = rpw // 128

    @functools.partial(
        pl.kernel, mesh=mesh,
        compiler_params=pltpu.CompilerParams(use_tc_tiling_on_sc=False),
        out_type=jax.ShapeDtypeStruct((rows_total, 16), jnp.float32),
        scratch_types=[
            pltpu.VMEM((nchunk, 128), jnp.int32),
            pltpu.VMEM((rpw, 16), jnp.float32),
            pltpu.VMEM((gpw, 16), jnp.float32),
            pltpu.SemaphoreType.DMA,
        ],
    )
    def sc_gather(xyz16_hbm, fidx_hbm, cattr16_hbm, out_hbm,
                  idx_v, rows_v, cent_v, sem):
        nc_ = plsc.get_sparse_core_info().num_cores
        wid = lax.axis_index("s") * nc_ + lax.axis_index("c")
        rbase = wid * rpw
        gbase = wid * gpw
        pltpu.sync_copy(fidx_hbm.at[wid], idx_v)
        pltpu.sync_copy(cattr16_hbm.at[pl.ds(gbase, gpw)], cent_v)
        copies = []
        for c in range(nchunk):
            copies.append(pltpu.async_copy(
                xyz16_hbm.at[idx_v.at[c]],
                rows_v.at[pl.ds(c * 128, 128)], sem))
        for cp in copies:
            cp.wait()

        lane = jnp.arange(16, dtype=jnp.int32)
        zero = jnp.zeros((16,), dtype=jnp.float32)

        def group_body(g, _):
            cmask = jnp.where(lane < 3, cent_v[g], zero)
            base = g * _S
            for s in range(_S):
                rows_v[base + s] = rows_v[base + s] - cmask
            return 0

        lax.fori_loop(0, gpw, group_body, 0)
        pltpu.sync_copy(rows_v, out_hbm.at[pl.ds(rbase, rpw)])

    return sc_gather


@jax.jit
def kernel(xyz):
    b, n, a = xyz.shape
    f32 = jnp.float32
    xyz8 = jnp.concatenate(
        [xyz, jnp.zeros((b, n, 8 - a), dtype=xyz.dtype)], axis=-1)
    xyzT = xyz8.transpose(0, 2, 1)
    planes = xyz[:, :, :3].transpose(0, 2, 1).reshape(b, 3, _NP, n // _NP)

    fidx, idx, cattr = pl.pallas_call(
        _tc_kernel,
        grid=(b,),
        in_specs=[
            pl.BlockSpec((1, n, 8), lambda i: (i, 0, 0)),
            pl.BlockSpec((1, 8, n), lambda i: (i, 0, 0)),
            pl.BlockSpec((1, _NP, n // _NP), lambda i: (i, 0, 0)),
            pl.BlockSpec((1, _NP, n // _NP), lambda i: (i, 0, 0)),
            pl.BlockSpec((1, _NP, n // _NP), lambda i: (i, 0, 0)),
        ],
        out_specs=[
            pl.BlockSpec((1, _G, _S), lambda i: (i, 0, 0)),
            pl.BlockSpec((1, 1, _G), lambda i: (i, 0, 0)),
            pl.BlockSpec((1, _G, 8), lambda i: (i, 0, 0)),
        ],
        out_shape=[
            jax.ShapeDtypeStruct((b, _G, _S), jnp.int32),
            jax.ShapeDtypeStruct((b, 1, _G), jnp.int32),
            jax.ShapeDtypeStruct((b, _G, 8), f32),
        ],
        scratch_shapes=[
            pltpu.VMEM((_G, n), f32),
            pltpu.VMEM((_G, 8), f32),
        ],
        compiler_params=pltpu.CompilerParams(
            dimension_semantics=("arbitrary",),
        ),
    )(xyz8, xyzT, planes[:, 0], planes[:, 1], planes[:, 2])

    rows_total = b * _G * _S
    nw = 32
    rpw = rows_total // nw
    gpw = (b * _G) // nw
    xyz16 = jnp.concatenate(
        [xyz.reshape(b * n, a),
         jnp.zeros((b * n, 16 - a), dtype=f32)], axis=-1)
    cattr16 = jnp.concatenate(
        [cattr.reshape(b * _G, 8),
         jnp.zeros((b * _G, 8), dtype=f32)], axis=-1)
    fidx3 = fidx.reshape(nw, rpw // 128, 128)

    nb16 = _sc_gather_make(rows_total, nw, gpw, rpw)(xyz16, fidx3, cattr16)

    neighborhood = nb16.reshape(b, _G, _S, 16)[..., :a]
    center_idx = idx.reshape(b, _G)
    centroids_attrs = cattr[..., :a]
    return (neighborhood, center_idx, centroids_attrs,
            centroids_attrs[..., :3])
